# baseline (speedup 1.0000x reference)
import jax
import jax.numpy as jnp
from jax import lax
from jax.experimental import pallas as pl
from jax.experimental.pallas import tpu as pltpu
from jax.experimental.pallas import tpu_sc as plsc

HASH_SIZE = 2 ** 20

_NC = 2
_NS = 16
_NW = _NC * _NS
_L = 16
_CHUNK = 128

_C1 = 2246822507
_C2 = 3266489909


def _u32(v):
    return jnp.uint32(v)


def _umulhi(a, b):
    b0 = _u32(b & 0xFFFF)
    b1 = _u32(b >> 16)
    a0 = a & _u32(0xFFFF)
    a1 = lax.shift_right_logical(a, _u32(16))
    t = a0 * b0
    m1 = a1 * b0 + lax.shift_right_logical(t, _u32(16))
    m2 = a0 * b1
    return (a1 * b1
            + lax.shift_right_logical(m1, _u32(16))
            + lax.shift_right_logical(m2, _u32(16))
            + lax.shift_right_logical((m1 & _u32(0xFFFF)) + (m2 & _u32(0xFFFF)),
                                      _u32(16)))


def _floor_i32(x):
    t = x.astype(jnp.int32)
    tf = t.astype(jnp.float32)
    return jnp.where(x < tf, t - 1, t)


def _sar64(hi, lo, k):
    s_lo = (lax.shift_right_logical(lo, _u32(k))
            | lax.shift_left(hi, _u32(32 - k)))
    s_hi = lax.bitcast_convert_type(
        lax.shift_right_arithmetic(
            lax.bitcast_convert_type(hi, jnp.int32), jnp.int32(k)),
        jnp.uint32)
    return s_hi, s_lo


def _hash16(x, y):
    ix = _floor_i32(x)
    iy = _floor_i32(y)
    lo = lax.bitcast_convert_type(ix, jnp.uint32)
    hi = lax.bitcast_convert_type(
        lax.shift_right_arithmetic(ix, jnp.int32(31)), jnp.uint32)
    s_hi, s_lo = _sar64(hi, lo, 16)
    lo = lo ^ s_lo
    hi = hi ^ s_hi
    new_lo = lo * _u32(_C1)
    hi = hi * _u32(_C1) + _umulhi(lo, _C1)
    lo = new_lo
    s_hi, s_lo = _sar64(hi, lo, 13)
    lo = lo ^ s_lo
    hi = hi ^ s_hi
    iy_lo = lax.bitcast_convert_type(iy, jnp.uint32)
    iy_hi = lax.bitcast_convert_type(
        lax.shift_right_arithmetic(iy, jnp.int32(31)), jnp.uint32)
    p_lo = iy_lo * _u32(_C2)
    p_hi = iy_hi * _u32(_C2) + _umulhi(iy_lo, _C2)
    sum_lo = lo + p_lo
    carry = jnp.where(sum_lo < lo, _u32(1), _u32(0))
    hi = hi + p_hi + carry
    lo = sum_lo
    _, s_lo = _sar64(hi, lo, 16)
    lo = lo ^ s_lo
    return lax.bitcast_convert_type(lo & _u32(HASH_SIZE - 1), jnp.int32)


def _make_kernel(n, nbuckets, dim):
    bpw = n // _NW
    epw = bpw * dim
    nstream = epw // _CHUNK
    bt_stride = 8 * 128
    ct_stride = (nbuckets // 128) * bt_stride
    out_ct_stride = (n // 128) * bt_stride
    nct = dim // 8
    nit = bpw // 128
    mesh = plsc.VectorSubcoreMesh(core_axis_name="c", subcore_axis_name="s")

    def body(xy_hbm, flat_hbm, out_hbm, xy_v, offs_v, rows_v, sem_a, sem_b):
        wid = lax.axis_index("s") * _NC + lax.axis_index("c")
        base = (wid * bpw).astype(jnp.int32)
        pltpu.sync_copy(xy_hbm.at[pl.ds(base * 2, 2 * bpw)], xy_v)
        for itl in range(nit):
            def hash_expand(q, carry, itl=itl):
                blk = jnp.int32(itl * 256) + q * _L
                b = _hash16(xy_v[pl.ds(blk, _L)], xy_v[pl.ds(blk + 128, _L)])
                bt = lax.shift_right_logical(b, jnp.int32(7))
                bv = lax.shift_left(bt, jnp.int32(10)) | (b & jnp.int32(127))
                dst0 = itl * 1024 + q * _L
                for ct in range(nct):
                    for ci in range(8):
                        offs_v[pl.ds(dst0 + ct * 4096 + ci * 128, _L)] = (
                            bv + jnp.int32(ct * ct_stride + ci * 128))
                return carry
            lax.fori_loop(jnp.int32(0), jnp.int32(128 // _L), hash_expand,
                          jnp.int32(0))
            for ct in range(nct):
                o = ct * 4096 + itl * 1024
                pltpu.async_copy(flat_hbm.at[offs_v.at[pl.ds(o, 1024)]],
                                 rows_v.at[pl.ds(o, 1024)],
                                 sem_a if ct < nct // 2 else sem_b)
        half = epw // 2
        pltpu.make_async_copy(
            flat_hbm.at[pl.ds(jnp.int32(0), half)],
            rows_v.at[pl.ds(0, half)], sem_a).wait()
        for ct in range(nct // 2):
            pltpu.async_copy(
                rows_v.at[pl.ds(ct * (nit * 1024), nit * 1024)],
                out_hbm.at[pl.ds(jnp.int32(ct * out_ct_stride) + wid * (nit * 1024),
                                 nit * 1024)], sem_a)
        pltpu.make_async_copy(
            flat_hbm.at[pl.ds(jnp.int32(0), half)],
            rows_v.at[pl.ds(half, half)], sem_b).wait()
        for ct in range(nct // 2, nct):
            pltpu.async_copy(
                rows_v.at[pl.ds(ct * (nit * 1024), nit * 1024)],
                out_hbm.at[pl.ds(jnp.int32(ct * out_ct_stride) + wid * (nit * 1024),
                                 nit * 1024)], sem_b)
        pltpu.make_async_copy(
            flat_hbm.at[pl.ds(jnp.int32(0), half)],
            rows_v.at[pl.ds(0, half)], sem_a).wait()
        pltpu.make_async_copy(
            flat_hbm.at[pl.ds(jnp.int32(0), half)],
            rows_v.at[pl.ds(half, half)], sem_b).wait()

    return pl.kernel(
        body,
        out_type=jax.ShapeDtypeStruct((n * dim,), jnp.float32),
        mesh=mesh,
        compiler_params=pltpu.CompilerParams(use_tc_tiling_on_sc=False),
        scratch_types=[
            pltpu.VMEM((2 * bpw,), jnp.float32),
            pltpu.VMEM((epw,), jnp.int32),
            pltpu.VMEM((epw,), jnp.float32),
            pltpu.SemaphoreType.DMA,
            pltpu.SemaphoreType.DMA,
        ],
    )


def kernel(positions, table):
    n = positions.shape[0]
    nbuckets, dim = table.shape
    xy = positions.reshape(n // 128, 128, 2).transpose(0, 2, 1).reshape(-1)
    flat = (table.T
            .reshape(dim // 8, 8, nbuckets // 128, 128)
            .transpose(0, 2, 1, 3)
            .reshape(-1))
    out = _make_kernel(n, nbuckets, dim)(xy, flat)
    return (out.reshape(dim // 8, n // 128, 8, 128)
            .transpose(1, 3, 0, 2)
            .reshape(n, dim))


# speedup vs baseline: 1.0033x; 1.0033x over previous
import jax
import jax.numpy as jnp
from jax import lax
from jax.experimental import pallas as pl
from jax.experimental.pallas import tpu as pltpu
from jax.experimental.pallas import tpu_sc as plsc

HASH_SIZE = 2 ** 20

_NC = 2
_NS = 16
_NW = _NC * _NS
_L = 16

_C1 = 2246822507
_C2 = 3266489909


def _u32(v):
    return jnp.uint32(v)


def _umulhi(a, b):
    b0 = _u32(b & 0xFFFF)
    b1 = _u32(b >> 16)
    a0 = a & _u32(0xFFFF)
    a1 = lax.shift_right_logical(a, _u32(16))
    t = a0 * b0
    m1 = a1 * b0 + lax.shift_right_logical(t, _u32(16))
    m2 = a0 * b1
    return (a1 * b1
            + lax.shift_right_logical(m1, _u32(16))
            + lax.shift_right_logical(m2, _u32(16))
            + lax.shift_right_logical((m1 & _u32(0xFFFF)) + (m2 & _u32(0xFFFF)),
                                      _u32(16)))


def _floor_i32(x):
    t = x.astype(jnp.int32)
    tf = t.astype(jnp.float32)
    return jnp.where(x < tf, t - 1, t)


def _sar64(hi, lo, k):
    s_lo = (lax.shift_right_logical(lo, _u32(k))
            | lax.shift_left(hi, _u32(32 - k)))
    s_hi = lax.bitcast_convert_type(
        lax.shift_right_arithmetic(
            lax.bitcast_convert_type(hi, jnp.int32), jnp.int32(k)),
        jnp.uint32)
    return s_hi, s_lo


def _hash16(x, y):
    ix = _floor_i32(x)
    iy = _floor_i32(y)
    lo = lax.bitcast_convert_type(ix, jnp.uint32)
    hi = lax.bitcast_convert_type(
        lax.shift_right_arithmetic(ix, jnp.int32(31)), jnp.uint32)
    s_hi, s_lo = _sar64(hi, lo, 16)
    lo = lo ^ s_lo
    hi = hi ^ s_hi
    new_lo = lo * _u32(_C1)
    hi = hi * _u32(_C1) + _umulhi(lo, _C1)
    lo = new_lo
    s_hi, s_lo = _sar64(hi, lo, 13)
    lo = lo ^ s_lo
    hi = hi ^ s_hi
    iy_lo = lax.bitcast_convert_type(iy, jnp.uint32)
    iy_hi = lax.bitcast_convert_type(
        lax.shift_right_arithmetic(iy, jnp.int32(31)), jnp.uint32)
    p_lo = iy_lo * _u32(_C2)
    p_hi = iy_hi * _u32(_C2) + _umulhi(iy_lo, _C2)
    sum_lo = lo + p_lo
    carry = jnp.where(sum_lo < lo, _u32(1), _u32(0))
    hi = hi + p_hi + carry
    lo = sum_lo
    _, s_lo = _sar64(hi, lo, 16)
    lo = lo ^ s_lo
    return lax.bitcast_convert_type(lo & _u32(HASH_SIZE - 1), jnp.int32)


def _make_kernel(n, nbuckets, dim):
    bpw = n // _NW
    epw = bpw * dim
    bt_stride = 8 * 128
    ct_stride = (nbuckets // 128) * bt_stride
    out_ct_stride = (n // 128) * bt_stride
    nct = dim // 8
    nit = bpw // 128
    mesh = plsc.VectorSubcoreMesh(core_axis_name="c", subcore_axis_name="s")

    def body(xy_hbm, flat_hbm, out_hbm, xy_v, offs_v, rows_v, sem):
        wid = lax.axis_index("s") * _NC + lax.axis_index("c")
        base = (wid * bpw).astype(jnp.int32)
        pltpu.sync_copy(xy_hbm.at[pl.ds(base * 2, 2 * bpw)], xy_v)
        for itl in range(nit):
            def hash_expand(q, carry, itl=itl):
                blk = jnp.int32(itl * 256) + q * _L
                b = _hash16(xy_v[pl.ds(blk, _L)], xy_v[pl.ds(blk + 128, _L)])
                bt = lax.shift_right_logical(b, jnp.int32(7))
                bv = lax.shift_left(bt, jnp.int32(10)) | (b & jnp.int32(127))
                dst0 = itl * 1024 + q * _L
                for ct in range(nct):
                    for ci in range(8):
                        offs_v[pl.ds(dst0 + ct * 4096 + ci * 128, _L)] = (
                            bv + jnp.int32(ct * ct_stride + ci * 128))
                return carry
            lax.fori_loop(jnp.int32(0), jnp.int32(128 // _L), hash_expand,
                          jnp.int32(0))
            for ct in range(nct):
                o = ct * 4096 + itl * 1024
                pltpu.async_copy(flat_hbm.at[offs_v.at[pl.ds(o, 1024)]],
                                 rows_v.at[pl.ds(o, 1024)], sem)
        pltpu.make_async_copy(
            flat_hbm.at[pl.ds(jnp.int32(0), epw)], rows_v, sem).wait()
        for ct in range(nct):
            pltpu.async_copy(
                rows_v.at[pl.ds(ct * (nit * 1024), nit * 1024)],
                out_hbm.at[pl.ds(jnp.int32(ct * out_ct_stride) + wid * (nit * 1024),
                                 nit * 1024)], sem)
        pltpu.make_async_copy(
            flat_hbm.at[pl.ds(jnp.int32(0), epw)], rows_v, sem).wait()

    return pl.kernel(
        body,
        out_type=jax.ShapeDtypeStruct((n * dim,), jnp.float32),
        mesh=mesh,
        compiler_params=pltpu.CompilerParams(use_tc_tiling_on_sc=False),
        scratch_types=[
            pltpu.VMEM((2 * bpw,), jnp.float32),
            pltpu.VMEM((epw,), jnp.int32),
            pltpu.VMEM((epw,), jnp.float32),
            pltpu.SemaphoreType.DMA,
        ],
    )


def kernel(positions, table):
    n = positions.shape[0]
    nbuckets, dim = table.shape
    xy = positions.reshape(n // 128, 128, 2).transpose(0, 2, 1).reshape(-1)
    flat = (table.T
            .reshape(dim // 8, 8, nbuckets // 128, 128)
            .transpose(0, 2, 1, 3)
            .reshape(-1))
    out = _make_kernel(n, nbuckets, dim)(xy, flat)
    return (out.reshape(dim // 8, n // 128, 8, 128)
            .transpose(1, 3, 0, 2)
            .reshape(n, dim))
